# baseline (speedup 1.0000x reference)
import functools
import jax
import jax.numpy as jnp
from jax import lax
from jax.experimental import pallas as pl
from jax.experimental.pallas import tpu as pltpu
from jax.experimental.pallas import tpu_sc as plsc


def _build_sc_gather(B, C, T, V, D):
    info = plsc.get_sparse_core_info()
    NC, NS, L = info.num_cores, info.num_subcores, info.num_lanes
    NW = NC * NS
    N = B * C * T
    per_w = N // NW
    CH = 512
    H = CH // 2
    n_chunks = per_w // CH
    NSLOT = 3
    LA = NSLOT - 1

    mesh = plsc.VectorSubcoreMesh(core_axis_name="c", subcore_axis_name="s")

    @functools.partial(
        pl.kernel,
        mesh=mesh,
        compiler_params=pltpu.CompilerParams(use_tc_tiling_on_sc=False),
        out_type=jax.ShapeDtypeStruct((N * D // 128, 128), jnp.float32),
        scratch_types=[
            pltpu.VMEM((CH,), jnp.int32),
            pltpu.VMEM((CH,), jnp.int32),
            pltpu.VMEM((CH,), jnp.int32),
            pltpu.VMEM((2, H, D), jnp.float32),
            pltpu.VMEM((2, H, D), jnp.float32),
            pltpu.VMEM((2, H, D), jnp.float32),
            pltpu.SemaphoreType.DMA((NSLOT,)),
            pltpu.SemaphoreType.DMA((NSLOT,)),
        ],
    )
    def k(idx_hbm, table_hbm, out_hbm, idx_v0, idx_v1, idx_v2,
          rows_v0, rows_v1, rows_v2, gsem, osem):
        idx_v = (idx_v0, idx_v1, idx_v2)
        rows_v = (rows_v0, rows_v1, rows_v2)
        wid = lax.axis_index("s") * NC + lax.axis_index("c")

        def gathers(slot):
            return tuple(
                pltpu.make_async_copy(
                    table_hbm.at[idx_v[slot].at[pl.ds(par * H, H)]],
                    rows_v[slot].at[par],
                    gsem.at[slot],
                )
                for par in (0, 1)
            )

        def load_and_gather(ci, slot):
            g = wid * n_chunks + ci
            start = g * CH
            c = (start // T) % C
            off = (c * (V // C)).astype(jnp.int32)
            iv = idx_v[slot]
            pltpu.sync_copy(idx_hbm.at[pl.ds(start, CH)], iv)

            def add_body(j, _):
                sl = pl.ds(j * L, L)
                iv[sl] = iv[sl] + off
                return 0

            lax.fori_loop(0, CH // L, add_body, 0, unroll=True)
            for cp in gathers(slot):
                cp.start()

        def out_slice(ci, par):
            start = (wid * n_chunks + ci) * CH
            return out_hbm.at[pl.ds(start // 2, H), pl.ds(par * D, D)]

        def gather_wait(slot):
            for cp in gathers(slot):
                cp.wait()

        def write_issue(ci, slot):
            for par in (0, 1):
                pltpu.async_copy(
                    rows_v[slot].at[par], out_slice(ci, par), osem.at[slot]
                )

        def write_wait(ci, slot):
            for par in (0, 1):
                pltpu.make_async_copy(
                    rows_v[slot].at[par], out_slice(ci, par), osem.at[slot]
                ).wait()

        for g in range(n_chunks + LA):
            if g < n_chunks:
                s = g % NSLOT
                if g >= NSLOT:
                    write_wait(g - NSLOT, s)
                load_and_gather(g, s)
            if g >= LA:
                gd = g - LA
                s = gd % NSLOT
                gather_wait(s)
                write_issue(gd, s)
        for gd in range(n_chunks - NSLOT, n_chunks):
            write_wait(gd, gd % NSLOT)

    return k


def kernel(input_ids, table):
    B_, C_, T_ = input_ids.shape
    V_, D_ = table.shape
    N_ = B_ * C_ * T_
    CH = 512
    flat_idx = (
        input_ids.reshape(N_ // CH, CH // 2, 2)
        .transpose(0, 2, 1)
        .reshape(N_)
        .astype(jnp.int32)
    )
    k = _build_sc_gather(B_, C_, T_, V_, D_)
    out2d = k(flat_idx, table)
    return out2d.reshape(B_, C_, T_, D_)


# speedup vs baseline: 1.0064x; 1.0064x over previous
import functools
import jax
import jax.numpy as jnp
from jax import lax
from jax.experimental import pallas as pl
from jax.experimental.pallas import tpu as pltpu
from jax.experimental.pallas import tpu_sc as plsc


def _build_sc_gather(B, C, T, V, D):
    info = plsc.get_sparse_core_info()
    NC, NS, L = info.num_cores, info.num_subcores, info.num_lanes
    NW = NC * NS
    N = B * C * T
    per_w = N // NW
    CH = 256
    H = CH // 2
    n_chunks = per_w // CH
    NSLOT = 7
    LA = NSLOT - 1

    mesh = plsc.VectorSubcoreMesh(core_axis_name="c", subcore_axis_name="s")

    @functools.partial(
        pl.kernel,
        mesh=mesh,
        compiler_params=pltpu.CompilerParams(use_tc_tiling_on_sc=False),
        out_type=jax.ShapeDtypeStruct((N * D // 128, 128), jnp.float32),
        scratch_types=(
            [pltpu.VMEM((CH,), jnp.int32)] * NSLOT
            + [pltpu.VMEM((2, H, D), jnp.float32)] * NSLOT
            + [
                pltpu.SemaphoreType.DMA((NSLOT,)),
                pltpu.SemaphoreType.DMA((NSLOT,)),
            ]
        ),
    )
    def k(idx_hbm, table_hbm, out_hbm, *scr):
        idx_v = scr[:NSLOT]
        rows_v = scr[NSLOT : 2 * NSLOT]
        gsem, osem = scr[2 * NSLOT :]
        wid = lax.axis_index("s") * NC + lax.axis_index("c")

        def gathers(slot):
            return tuple(
                pltpu.make_async_copy(
                    table_hbm.at[idx_v[slot].at[pl.ds(par * H, H)]],
                    rows_v[slot].at[par],
                    gsem.at[slot],
                )
                for par in (0, 1)
            )

        def load_and_gather(ci, slot):
            g = wid * n_chunks + ci
            start = g * CH
            c = (start // T) % C
            off = (c * (V // C)).astype(jnp.int32)
            iv = idx_v[slot]
            pltpu.sync_copy(idx_hbm.at[pl.ds(start, CH)], iv)

            def add_body(j, _):
                sl = pl.ds(j * L, L)
                iv[sl] = iv[sl] + off
                return 0

            lax.fori_loop(0, CH // L, add_body, 0, unroll=True)
            for cp in gathers(slot):
                cp.start()

        def out_slice(ci, par):
            start = (wid * n_chunks + ci) * CH
            return out_hbm.at[pl.ds(start // 2, H), pl.ds(par * D, D)]

        def gather_wait(slot):
            for cp in gathers(slot):
                cp.wait()

        def write_issue(ci, slot):
            for par in (0, 1):
                pltpu.async_copy(
                    rows_v[slot].at[par], out_slice(ci, par), osem.at[slot]
                )

        def write_wait(ci, slot):
            for par in (0, 1):
                pltpu.make_async_copy(
                    rows_v[slot].at[par], out_slice(ci, par), osem.at[slot]
                ).wait()

        for g in range(n_chunks + LA):
            if g < n_chunks:
                s = g % NSLOT
                if g >= NSLOT:
                    write_wait(g - NSLOT, s)
                load_and_gather(g, s)
            if g >= LA:
                gd = g - LA
                s = gd % NSLOT
                gather_wait(s)
                write_issue(gd, s)
        for gd in range(n_chunks - NSLOT, n_chunks):
            write_wait(gd, gd % NSLOT)

    return k


def kernel(input_ids, table):
    B_, C_, T_ = input_ids.shape
    V_, D_ = table.shape
    N_ = B_ * C_ * T_
    CH = 256
    flat_idx = (
        input_ids.reshape(N_ // CH, CH // 2, 2)
        .transpose(0, 2, 1)
        .reshape(N_)
        .astype(jnp.int32)
    )
    k = _build_sc_gather(B_, C_, T_, V_, D_)
    out2d = k(flat_idx, table)
    return out2d.reshape(B_, C_, T_, D_)
